# baseline (speedup 1.0000x reference)
import functools

import jax
import jax.numpy as jnp
from jax import lax
from jax.experimental import pallas as pl
from jax.experimental.pallas import tpu as pltpu
from jax.experimental.pallas import tpu_sc as plsc

N_NODES = 10000
D = 128
N_EDGES = 320000

N_PAD = 10240
NC, NS = 2, 16
NW = NC * NS
CHUNK = 128
CHUNKS_PER_TILE = 80
E_PAD = NW * CHUNKS_PER_TILE * CHUNK
ROWS_PER_TILE = N_PAD // NS

_mesh = plsc.VectorSubcoreMesh(
    core_axis_name="c", subcore_axis_name="s", num_cores=NC, num_subcores=NS)


@functools.partial(
    pl.kernel,
    out_type=jax.ShapeDtypeStruct((NC, N_PAD, D), jnp.float32),
    mesh=_mesh,
    scratch_types=[
        pltpu.MemorySpace.VMEM((CHUNKS_PER_TILE, CHUNK), jnp.int32),
        pltpu.MemorySpace.VMEM((CHUNK, D), jnp.float32),
        pltpu.MemorySpace.VMEM_SHARED((N_PAD, D), jnp.float32),
        pltpu.SemaphoreType.DMA,
    ],
)
def _deg_pass(dst_hbm, zeros_hbm, ones_hbm, cnt_hbm, dst_v, ones_v, acc_sh, sem):
    c = lax.axis_index("c")
    s = lax.axis_index("s")
    wid = s * NC + c
    pltpu.sync_copy(zeros_hbm, acc_sh.at[pl.ds(s * ROWS_PER_TILE, ROWS_PER_TILE)])
    pltpu.sync_copy(ones_hbm, ones_v)
    pltpu.sync_copy(dst_hbm.at[pl.ds(wid * CHUNKS_PER_TILE, CHUNKS_PER_TILE)], dst_v)
    plsc.subcore_barrier()

    def body(j, _):
        pltpu.sync_copy(ones_v, acc_sh.at[dst_v.at[j]], add=True)
        return 0

    lax.fori_loop(0, CHUNKS_PER_TILE, body, 0)

    plsc.subcore_barrier()
    pltpu.sync_copy(
        acc_sh.at[pl.ds(s * ROWS_PER_TILE, ROWS_PER_TILE)],
        cnt_hbm.at[c, pl.ds(s * ROWS_PER_TILE, ROWS_PER_TILE)],
    )


@functools.partial(
    pl.kernel,
    out_type=jax.ShapeDtypeStruct((NC, N_PAD, D), jnp.float32),
    mesh=_mesh,
    scratch_types=[
        pltpu.MemorySpace.VMEM((CHUNKS_PER_TILE // 2, CHUNK), jnp.int32),
        pltpu.MemorySpace.VMEM((CHUNKS_PER_TILE // 2, CHUNK), jnp.int32),
        pltpu.MemorySpace.VMEM((2 * CHUNK, D), jnp.float32),
        pltpu.MemorySpace.VMEM_SHARED((N_PAD, D), jnp.float32),
        pltpu.SemaphoreType.DMA,
        pltpu.SemaphoreType.DMA,
        pltpu.SemaphoreType.DMA,
        pltpu.SemaphoreType.DMA,
        pltpu.SemaphoreType.DMA,
        pltpu.SemaphoreType.DMA,
    ],
)
def _msg_pass(y_hbm, src_hbm, dst_hbm, zeros_hbm, out_hbm,
              src_v, dst_v, rows_v, acc_sh,
              g0, g1, g2, g3, s0, s1):
    c = lax.axis_index("c")
    s = lax.axis_index("s")
    wid = s * NC + c
    gsem = [g0, g1, g2, g3]
    ssem = [s0, s1]
    half = CHUNKS_PER_TILE // 2
    pltpu.sync_copy(zeros_hbm, acc_sh.at[pl.ds(s * ROWS_PER_TILE, ROWS_PER_TILE)])
    plsc.subcore_barrier()

    for h in range(2):
        base = wid * CHUNKS_PER_TILE + h * half
        pltpu.sync_copy(src_hbm.at[pl.ds(base, half)], src_v)
        pltpu.sync_copy(dst_hbm.at[pl.ds(base, half)], dst_v)

        def body(t, _):
            jj = 2 * t
            gd = [
                pltpu.async_copy(
                    y_hbm.at[src_v.at[jj + q // 2, pl.ds((q % 2) * 64, 64)]],
                    rows_v.at[pl.ds(q * 64, 64)],
                    gsem[q])
                for q in range(4)
            ]
            sd = []
            for m in range(2):
                gd[2 * m].wait()
                gd[2 * m + 1].wait()
                sd.append(pltpu.async_copy(
                    rows_v.at[pl.ds(m * CHUNK, CHUNK)],
                    acc_sh.at[dst_v.at[jj + m]], ssem[m], add=True))
            for m in range(2):
                sd[m].wait()
            return 0

        lax.fori_loop(0, half // 2, body, 0)
    plsc.subcore_barrier()
    pltpu.sync_copy(
        acc_sh.at[pl.ds(s * ROWS_PER_TILE, ROWS_PER_TILE)],
        out_hbm.at[c, pl.ds(s * ROWS_PER_TILE, ROWS_PER_TILE)],
    )


_BLK = 256
_GRID = N_PAD // _BLK


def _mm_body(x_ref, w_ref, o_ref):
    o_ref[...] = jnp.dot(x_ref[...], w_ref[...], preferred_element_type=jnp.float32)


_mm = pl.pallas_call(
    _mm_body,
    grid=(_GRID,),
    in_specs=[
        pl.BlockSpec((_BLK, D), lambda i: (i, 0)),
        pl.BlockSpec((D, D), lambda i: (0, 0)),
    ],
    out_specs=pl.BlockSpec((_BLK, D), lambda i: (i, 0)),
    out_shape=jax.ShapeDtypeStruct((N_PAD, D), jnp.float32),
)


def _dinv_y_body(c0_ref, c1_ref, t_ref, dinv_ref, y_ref):
    i = pl.program_id(0)
    cnt = c0_ref[:, 0:1] + c1_ref[:, 0:1]
    deg = cnt + 1.0
    dinv = lax.rsqrt(deg)
    row = i * _BLK + lax.broadcasted_iota(jnp.int32, (_BLK, 1), 0)
    dinv = jnp.where(row < N_NODES, dinv, 0.0)
    dinv_b = jnp.broadcast_to(dinv, (_BLK, D))
    dinv_ref[...] = dinv_b
    y_ref[...] = dinv_b * t_ref[...]


_dinv_y = pl.pallas_call(
    _dinv_y_body,
    grid=(_GRID,),
    in_specs=[
        pl.BlockSpec((_BLK, D), lambda i: (i, 0)),
        pl.BlockSpec((_BLK, D), lambda i: (i, 0)),
        pl.BlockSpec((_BLK, D), lambda i: (i, 0)),
    ],
    out_specs=[
        pl.BlockSpec((_BLK, D), lambda i: (i, 0)),
        pl.BlockSpec((_BLK, D), lambda i: (i, 0)),
    ],
    out_shape=[
        jax.ShapeDtypeStruct((N_PAD, D), jnp.float32),
        jax.ShapeDtypeStruct((N_PAD, D), jnp.float32),
    ],
)


def _fuse_body(p0_ref, p1_ref, y_ref, dinv_ref, b_ref, w_ref, yn_ref):
    h = dinv_ref[...] * (p0_ref[...] + p1_ref[...] + y_ref[...]) + b_ref[...]
    h = jnp.maximum(h, 0.0)
    t = jnp.dot(h, w_ref[...], preferred_element_type=jnp.float32)
    yn_ref[...] = dinv_ref[...] * t


_fuse = pl.pallas_call(
    _fuse_body,
    grid=(_GRID,),
    in_specs=[
        pl.BlockSpec((_BLK, D), lambda i: (i, 0)),
        pl.BlockSpec((_BLK, D), lambda i: (i, 0)),
        pl.BlockSpec((_BLK, D), lambda i: (i, 0)),
        pl.BlockSpec((_BLK, D), lambda i: (i, 0)),
        pl.BlockSpec((1, D), lambda i: (0, 0)),
        pl.BlockSpec((D, D), lambda i: (0, 0)),
    ],
    out_specs=pl.BlockSpec((_BLK, D), lambda i: (i, 0)),
    out_shape=jax.ShapeDtypeStruct((N_PAD, D), jnp.float32),
)


def _epi_body(p0_ref, p1_ref, y_ref, dinv_ref, b_ref, o_ref):
    o_ref[...] = (
        dinv_ref[...] * (p0_ref[...] + p1_ref[...] + y_ref[...]) + b_ref[...]
    )


_epi = pl.pallas_call(
    _epi_body,
    grid=(_GRID,),
    in_specs=[
        pl.BlockSpec((_BLK, D), lambda i: (i, 0)),
        pl.BlockSpec((_BLK, D), lambda i: (i, 0)),
        pl.BlockSpec((_BLK, D), lambda i: (i, 0)),
        pl.BlockSpec((_BLK, D), lambda i: (i, 0)),
        pl.BlockSpec((1, D), lambda i: (0, 0)),
    ],
    out_specs=pl.BlockSpec((_BLK, D), lambda i: (i, 0)),
    out_shape=jax.ShapeDtypeStruct((N_PAD, D), jnp.float32),
)


@jax.jit
def kernel(x, edge_index, W1, b1, W2, b2, W3, b3):
    src = edge_index[0]
    dst = edge_index[1]
    pad_e = E_PAD - N_EDGES
    src2 = jnp.concatenate(
        [src, jnp.full((pad_e,), N_NODES, jnp.int32)]).reshape(E_PAD // CHUNK, CHUNK)
    dst2 = jnp.concatenate(
        [dst, jnp.full((pad_e,), N_NODES, jnp.int32)]).reshape(E_PAD // CHUNK, CHUNK)
    x_pad = jnp.concatenate(
        [x, jnp.zeros((N_PAD - N_NODES, D), jnp.float32)], axis=0)

    zeros128 = jnp.zeros((ROWS_PER_TILE, D), jnp.float32)
    ones128 = jnp.ones((CHUNK, D), jnp.float32)
    b1r = b1.reshape(1, D)
    b2r = b2.reshape(1, D)
    b3r = b3.reshape(1, D)

    cnt = _deg_pass(dst2, zeros128, ones128)
    t1 = _mm(x_pad, W1)
    dinv, y1 = _dinv_y(cnt[0], cnt[1], t1)
    p = _msg_pass(y1, src2, dst2, zeros128)
    y2 = _fuse(p[0], p[1], y1, dinv, b1r, W2)
    p = _msg_pass(y2, src2, dst2, zeros128)
    y3 = _fuse(p[0], p[1], y2, dinv, b2r, W3)
    p = _msg_pass(y3, src2, dst2, zeros128)
    out = _epi(p[0], p[1], y3, dinv, b3r)
    return out[:N_NODES]


# speedup vs baseline: 1.0005x; 1.0005x over previous
import functools

import jax
import jax.numpy as jnp
from jax import lax
from jax.experimental import pallas as pl
from jax.experimental.pallas import tpu as pltpu
from jax.experimental.pallas import tpu_sc as plsc

N_NODES = 10000
D = 128
N_EDGES = 320000

N_PAD = 10240
NC, NS = 2, 16
NW = NC * NS
CHUNK = 128
CHUNKS_PER_TILE = 80
E_PAD = NW * CHUNKS_PER_TILE * CHUNK
ROWS_PER_TILE = N_PAD // NS

_mesh = plsc.VectorSubcoreMesh(
    core_axis_name="c", subcore_axis_name="s", num_cores=NC, num_subcores=NS)


@functools.partial(
    pl.kernel,
    out_type=jax.ShapeDtypeStruct((NC, N_PAD, D), jnp.float32),
    mesh=_mesh,
    scratch_types=[
        pltpu.MemorySpace.VMEM((CHUNKS_PER_TILE, CHUNK), jnp.int32),
        pltpu.MemorySpace.VMEM((CHUNK, D), jnp.float32),
        pltpu.MemorySpace.VMEM_SHARED((N_PAD, D), jnp.float32),
        pltpu.SemaphoreType.DMA,
    ],
)
def _deg_pass(dst_hbm, zeros_hbm, ones_hbm, cnt_hbm, dst_v, ones_v, acc_sh, sem):
    c = lax.axis_index("c")
    s = lax.axis_index("s")
    wid = s * NC + c
    pltpu.sync_copy(zeros_hbm, acc_sh.at[pl.ds(s * ROWS_PER_TILE, ROWS_PER_TILE)])
    pltpu.sync_copy(ones_hbm, ones_v)
    pltpu.sync_copy(dst_hbm.at[pl.ds(wid * CHUNKS_PER_TILE, CHUNKS_PER_TILE)], dst_v)
    plsc.subcore_barrier()

    def body(t, _):
        jj = 8 * t
        descs = [
            pltpu.async_copy(ones_v, acc_sh.at[dst_v.at[jj + m]], sem, add=True)
            for m in range(8)
        ]
        for d in descs:
            d.wait()
        return 0

    lax.fori_loop(0, CHUNKS_PER_TILE // 8, body, 0)

    plsc.subcore_barrier()
    pltpu.sync_copy(
        acc_sh.at[pl.ds(s * ROWS_PER_TILE, ROWS_PER_TILE)],
        cnt_hbm.at[c, pl.ds(s * ROWS_PER_TILE, ROWS_PER_TILE)],
    )


@functools.partial(
    pl.kernel,
    out_type=jax.ShapeDtypeStruct((NC, N_PAD, D), jnp.float32),
    mesh=_mesh,
    scratch_types=[
        pltpu.MemorySpace.VMEM((CHUNKS_PER_TILE // 2, CHUNK), jnp.int32),
        pltpu.MemorySpace.VMEM((CHUNKS_PER_TILE // 2, CHUNK), jnp.int32),
        pltpu.MemorySpace.VMEM((2 * CHUNK, D), jnp.float32),
        pltpu.MemorySpace.VMEM_SHARED((N_PAD, D), jnp.float32),
        pltpu.SemaphoreType.DMA,
        pltpu.SemaphoreType.DMA,
        pltpu.SemaphoreType.DMA,
        pltpu.SemaphoreType.DMA,
        pltpu.SemaphoreType.DMA,
        pltpu.SemaphoreType.DMA,
    ],
)
def _msg_pass(y_hbm, src_hbm, dst_hbm, zeros_hbm, out_hbm,
              src_v, dst_v, rows_v, acc_sh,
              g0, g1, g2, g3, s0, s1):
    c = lax.axis_index("c")
    s = lax.axis_index("s")
    wid = s * NC + c
    gsem = [g0, g1, g2, g3]
    ssem = [s0, s1]
    half = CHUNKS_PER_TILE // 2
    pltpu.sync_copy(zeros_hbm, acc_sh.at[pl.ds(s * ROWS_PER_TILE, ROWS_PER_TILE)])
    plsc.subcore_barrier()

    for h in range(2):
        base = wid * CHUNKS_PER_TILE + h * half
        pltpu.sync_copy(src_hbm.at[pl.ds(base, half)], src_v)
        pltpu.sync_copy(dst_hbm.at[pl.ds(base, half)], dst_v)

        def body(t, _):
            jj = 2 * t
            gd = [
                pltpu.async_copy(
                    y_hbm.at[src_v.at[jj + q // 2, pl.ds((q % 2) * 64, 64)]],
                    rows_v.at[pl.ds(q * 64, 64)],
                    gsem[q])
                for q in range(4)
            ]
            sd = []
            for m in range(2):
                gd[2 * m].wait()
                gd[2 * m + 1].wait()
                sd.append(pltpu.async_copy(
                    rows_v.at[pl.ds(m * CHUNK, CHUNK)],
                    acc_sh.at[dst_v.at[jj + m]], ssem[m], add=True))
            for m in range(2):
                sd[m].wait()
            return 0

        lax.fori_loop(0, half // 2, body, 0)
    plsc.subcore_barrier()
    pltpu.sync_copy(
        acc_sh.at[pl.ds(s * ROWS_PER_TILE, ROWS_PER_TILE)],
        out_hbm.at[c, pl.ds(s * ROWS_PER_TILE, ROWS_PER_TILE)],
    )


_BLK = 256
_GRID = N_PAD // _BLK


def _mm_body(x_ref, w_ref, o_ref):
    o_ref[...] = jnp.dot(x_ref[...], w_ref[...], preferred_element_type=jnp.float32)


_mm = pl.pallas_call(
    _mm_body,
    grid=(_GRID,),
    in_specs=[
        pl.BlockSpec((_BLK, D), lambda i: (i, 0)),
        pl.BlockSpec((D, D), lambda i: (0, 0)),
    ],
    out_specs=pl.BlockSpec((_BLK, D), lambda i: (i, 0)),
    out_shape=jax.ShapeDtypeStruct((N_PAD, D), jnp.float32),
)


def _dinv_y_body(c0_ref, c1_ref, t_ref, dinv_ref, y_ref):
    i = pl.program_id(0)
    cnt = c0_ref[:, 0:1] + c1_ref[:, 0:1]
    deg = cnt + 1.0
    dinv = lax.rsqrt(deg)
    row = i * _BLK + lax.broadcasted_iota(jnp.int32, (_BLK, 1), 0)
    dinv = jnp.where(row < N_NODES, dinv, 0.0)
    dinv_b = jnp.broadcast_to(dinv, (_BLK, D))
    dinv_ref[...] = dinv_b
    y_ref[...] = dinv_b * t_ref[...]


_dinv_y = pl.pallas_call(
    _dinv_y_body,
    grid=(_GRID,),
    in_specs=[
        pl.BlockSpec((_BLK, D), lambda i: (i, 0)),
        pl.BlockSpec((_BLK, D), lambda i: (i, 0)),
        pl.BlockSpec((_BLK, D), lambda i: (i, 0)),
    ],
    out_specs=[
        pl.BlockSpec((_BLK, D), lambda i: (i, 0)),
        pl.BlockSpec((_BLK, D), lambda i: (i, 0)),
    ],
    out_shape=[
        jax.ShapeDtypeStruct((N_PAD, D), jnp.float32),
        jax.ShapeDtypeStruct((N_PAD, D), jnp.float32),
    ],
)


def _fuse_body(p0_ref, p1_ref, y_ref, dinv_ref, b_ref, w_ref, yn_ref):
    h = dinv_ref[...] * (p0_ref[...] + p1_ref[...] + y_ref[...]) + b_ref[...]
    h = jnp.maximum(h, 0.0)
    t = jnp.dot(h, w_ref[...], preferred_element_type=jnp.float32)
    yn_ref[...] = dinv_ref[...] * t


_fuse = pl.pallas_call(
    _fuse_body,
    grid=(_GRID,),
    in_specs=[
        pl.BlockSpec((_BLK, D), lambda i: (i, 0)),
        pl.BlockSpec((_BLK, D), lambda i: (i, 0)),
        pl.BlockSpec((_BLK, D), lambda i: (i, 0)),
        pl.BlockSpec((_BLK, D), lambda i: (i, 0)),
        pl.BlockSpec((1, D), lambda i: (0, 0)),
        pl.BlockSpec((D, D), lambda i: (0, 0)),
    ],
    out_specs=pl.BlockSpec((_BLK, D), lambda i: (i, 0)),
    out_shape=jax.ShapeDtypeStruct((N_PAD, D), jnp.float32),
)


def _epi_body(p0_ref, p1_ref, y_ref, dinv_ref, b_ref, o_ref):
    o_ref[...] = (
        dinv_ref[...] * (p0_ref[...] + p1_ref[...] + y_ref[...]) + b_ref[...]
    )


_epi = pl.pallas_call(
    _epi_body,
    grid=(_GRID,),
    in_specs=[
        pl.BlockSpec((_BLK, D), lambda i: (i, 0)),
        pl.BlockSpec((_BLK, D), lambda i: (i, 0)),
        pl.BlockSpec((_BLK, D), lambda i: (i, 0)),
        pl.BlockSpec((_BLK, D), lambda i: (i, 0)),
        pl.BlockSpec((1, D), lambda i: (0, 0)),
    ],
    out_specs=pl.BlockSpec((_BLK, D), lambda i: (i, 0)),
    out_shape=jax.ShapeDtypeStruct((N_PAD, D), jnp.float32),
)


@jax.jit
def kernel(x, edge_index, W1, b1, W2, b2, W3, b3):
    src = edge_index[0]
    dst = edge_index[1]
    pad_e = E_PAD - N_EDGES
    src2 = jnp.concatenate(
        [src, jnp.full((pad_e,), N_NODES, jnp.int32)]).reshape(E_PAD // CHUNK, CHUNK)
    dst2 = jnp.concatenate(
        [dst, jnp.full((pad_e,), N_NODES, jnp.int32)]).reshape(E_PAD // CHUNK, CHUNK)
    x_pad = jnp.concatenate(
        [x, jnp.zeros((N_PAD - N_NODES, D), jnp.float32)], axis=0)

    zeros128 = jnp.zeros((ROWS_PER_TILE, D), jnp.float32)
    ones128 = jnp.ones((CHUNK, D), jnp.float32)
    b1r = b1.reshape(1, D)
    b2r = b2.reshape(1, D)
    b3r = b3.reshape(1, D)

    cnt = _deg_pass(dst2, zeros128, ones128)
    t1 = _mm(x_pad, W1)
    dinv, y1 = _dinv_y(cnt[0], cnt[1], t1)
    p = _msg_pass(y1, src2, dst2, zeros128)
    y2 = _fuse(p[0], p[1], y1, dinv, b1r, W2)
    p = _msg_pass(y2, src2, dst2, zeros128)
    y3 = _fuse(p[0], p[1], y2, dinv, b2r, W3)
    p = _msg_pass(y3, src2, dst2, zeros128)
    out = _epi(p[0], p[1], y3, dinv, b3r)
    return out[:N_NODES]


# speedup vs baseline: 1.0433x; 1.0428x over previous
import functools

import jax
import jax.numpy as jnp
from jax import lax
from jax.experimental import pallas as pl
from jax.experimental.pallas import tpu as pltpu
from jax.experimental.pallas import tpu_sc as plsc

N_NODES = 10000
D = 128
N_EDGES = 320000

N_PAD = 10240
NC, NS = 2, 16
NW = NC * NS
CHUNK = 128
CHUNKS_PER_TILE = 80
E_PAD = NW * CHUNKS_PER_TILE * CHUNK
ROWS_PER_TILE = N_PAD // NS

_mesh = plsc.VectorSubcoreMesh(
    core_axis_name="c", subcore_axis_name="s", num_cores=NC, num_subcores=NS)


@functools.partial(
    pl.kernel,
    out_type=jax.ShapeDtypeStruct((NC, N_PAD, D), jnp.float32),
    mesh=_mesh,
    scratch_types=[
        pltpu.MemorySpace.VMEM((CHUNKS_PER_TILE, CHUNK), jnp.int32),
        pltpu.MemorySpace.VMEM((CHUNK, D), jnp.float32),
        pltpu.MemorySpace.VMEM_SHARED((N_PAD, D), jnp.float32),
        pltpu.SemaphoreType.DMA,
    ],
)
def _deg_pass(dst_hbm, zeros_hbm, ones_hbm, cnt_hbm, dst_v, ones_v, acc_sh, sem):
    c = lax.axis_index("c")
    s = lax.axis_index("s")
    wid = s * NC + c
    pltpu.sync_copy(zeros_hbm, acc_sh.at[pl.ds(s * ROWS_PER_TILE, ROWS_PER_TILE)])
    pltpu.sync_copy(ones_hbm, ones_v)
    pltpu.sync_copy(dst_hbm.at[pl.ds(wid * CHUNKS_PER_TILE, CHUNKS_PER_TILE)], dst_v)
    plsc.subcore_barrier()

    def body(t, _):
        jj = 8 * t
        descs = [
            pltpu.async_copy(ones_v, acc_sh.at[dst_v.at[jj + m]], sem, add=True)
            for m in range(8)
        ]
        for d in descs:
            d.wait()
        return 0

    lax.fori_loop(0, CHUNKS_PER_TILE // 8, body, 0)

    plsc.subcore_barrier()
    pltpu.sync_copy(
        acc_sh.at[pl.ds(s * ROWS_PER_TILE, ROWS_PER_TILE)],
        cnt_hbm.at[c, pl.ds(s * ROWS_PER_TILE, ROWS_PER_TILE)],
    )


@functools.partial(
    pl.kernel,
    out_type=jax.ShapeDtypeStruct((NC, N_PAD, D), jnp.float32),
    mesh=_mesh,
    scratch_types=[
        pltpu.MemorySpace.VMEM((CHUNKS_PER_TILE // 2, CHUNK), jnp.int32),
        pltpu.MemorySpace.VMEM((CHUNKS_PER_TILE // 2, CHUNK), jnp.int32),
        pltpu.MemorySpace.VMEM((2 * CHUNK, D), jnp.float32),
        pltpu.MemorySpace.VMEM_SHARED((N_PAD, D), jnp.float32),
        pltpu.SemaphoreType.DMA,
        pltpu.SemaphoreType.DMA,
        pltpu.SemaphoreType.DMA,
        pltpu.SemaphoreType.DMA,
        pltpu.SemaphoreType.DMA,
        pltpu.SemaphoreType.DMA,
    ],
)
def _msg_pass(y_hbm, src_hbm, dst_hbm, zeros_hbm, out_hbm,
              src_v, dst_v, rows_v, acc_sh,
              g0, g1, g2, g3, s0, s1):
    c = lax.axis_index("c")
    s = lax.axis_index("s")
    wid = s * NC + c
    gsem = [g0, g1, g2, g3]
    ssem = [s0, s1]
    half = CHUNKS_PER_TILE // 2
    pltpu.sync_copy(zeros_hbm, acc_sh.at[pl.ds(s * ROWS_PER_TILE, ROWS_PER_TILE)])
    plsc.subcore_barrier()

    def issue_gather(j, b):
        pltpu.async_copy(y_hbm.at[src_v.at[j]],
                         rows_v.at[pl.ds(b * CHUNK, CHUNK)], gsem[b])

    def wait_gather(b):
        pltpu.make_async_copy(y_hbm.at[src_v.at[0]],
                              rows_v.at[pl.ds(b * CHUNK, CHUNK)], gsem[b]).wait()

    def issue_scatter(j, b):
        pltpu.async_copy(rows_v.at[pl.ds(b * CHUNK, CHUNK)],
                         acc_sh.at[dst_v.at[j]], ssem[b], add=True)

    def wait_scatter(b):
        pltpu.make_async_copy(rows_v.at[pl.ds(b * CHUNK, CHUNK)],
                              acc_sh.at[dst_v.at[0]], ssem[b]).wait()

    for h in range(2):
        base = wid * CHUNKS_PER_TILE + h * half
        pltpu.sync_copy(src_hbm.at[pl.ds(base, half)], src_v)
        pltpu.sync_copy(dst_hbm.at[pl.ds(base, half)], dst_v)

        issue_gather(0, 0)
        wait_gather(0)
        issue_scatter(0, 0)
        issue_gather(1, 1)
        wait_gather(1)
        issue_scatter(1, 1)

        def body(t, _):
            jj = 2 * t
            wait_scatter(0)
            issue_gather(jj, 0)
            wait_scatter(1)
            wait_gather(0)
            issue_scatter(jj, 0)
            issue_gather(jj + 1, 1)
            wait_gather(1)
            issue_scatter(jj + 1, 1)
            return 0

        lax.fori_loop(1, half // 2, body, 0)
        wait_scatter(0)
        wait_scatter(1)
    plsc.subcore_barrier()
    pltpu.sync_copy(
        acc_sh.at[pl.ds(s * ROWS_PER_TILE, ROWS_PER_TILE)],
        out_hbm.at[c, pl.ds(s * ROWS_PER_TILE, ROWS_PER_TILE)],
    )


_BLK = 256
_GRID = N_PAD // _BLK


def _mm_body(x_ref, w_ref, o_ref):
    o_ref[...] = jnp.dot(x_ref[...], w_ref[...], preferred_element_type=jnp.float32)


_mm = pl.pallas_call(
    _mm_body,
    grid=(_GRID,),
    in_specs=[
        pl.BlockSpec((_BLK, D), lambda i: (i, 0)),
        pl.BlockSpec((D, D), lambda i: (0, 0)),
    ],
    out_specs=pl.BlockSpec((_BLK, D), lambda i: (i, 0)),
    out_shape=jax.ShapeDtypeStruct((N_PAD, D), jnp.float32),
)


def _dinv_y_body(c0_ref, c1_ref, t_ref, dinv_ref, y_ref):
    i = pl.program_id(0)
    cnt = c0_ref[:, 0:1] + c1_ref[:, 0:1]
    deg = cnt + 1.0
    dinv = lax.rsqrt(deg)
    row = i * _BLK + lax.broadcasted_iota(jnp.int32, (_BLK, 1), 0)
    dinv = jnp.where(row < N_NODES, dinv, 0.0)
    dinv_b = jnp.broadcast_to(dinv, (_BLK, D))
    dinv_ref[...] = dinv_b
    y_ref[...] = dinv_b * t_ref[...]


_dinv_y = pl.pallas_call(
    _dinv_y_body,
    grid=(_GRID,),
    in_specs=[
        pl.BlockSpec((_BLK, D), lambda i: (i, 0)),
        pl.BlockSpec((_BLK, D), lambda i: (i, 0)),
        pl.BlockSpec((_BLK, D), lambda i: (i, 0)),
    ],
    out_specs=[
        pl.BlockSpec((_BLK, D), lambda i: (i, 0)),
        pl.BlockSpec((_BLK, D), lambda i: (i, 0)),
    ],
    out_shape=[
        jax.ShapeDtypeStruct((N_PAD, D), jnp.float32),
        jax.ShapeDtypeStruct((N_PAD, D), jnp.float32),
    ],
)


def _fuse_body(p0_ref, p1_ref, y_ref, dinv_ref, b_ref, w_ref, yn_ref):
    h = dinv_ref[...] * (p0_ref[...] + p1_ref[...] + y_ref[...]) + b_ref[...]
    h = jnp.maximum(h, 0.0)
    t = jnp.dot(h, w_ref[...], preferred_element_type=jnp.float32)
    yn_ref[...] = dinv_ref[...] * t


_fuse = pl.pallas_call(
    _fuse_body,
    grid=(_GRID,),
    in_specs=[
        pl.BlockSpec((_BLK, D), lambda i: (i, 0)),
        pl.BlockSpec((_BLK, D), lambda i: (i, 0)),
        pl.BlockSpec((_BLK, D), lambda i: (i, 0)),
        pl.BlockSpec((_BLK, D), lambda i: (i, 0)),
        pl.BlockSpec((1, D), lambda i: (0, 0)),
        pl.BlockSpec((D, D), lambda i: (0, 0)),
    ],
    out_specs=pl.BlockSpec((_BLK, D), lambda i: (i, 0)),
    out_shape=jax.ShapeDtypeStruct((N_PAD, D), jnp.float32),
)


def _epi_body(p0_ref, p1_ref, y_ref, dinv_ref, b_ref, o_ref):
    o_ref[...] = (
        dinv_ref[...] * (p0_ref[...] + p1_ref[...] + y_ref[...]) + b_ref[...]
    )


_epi = pl.pallas_call(
    _epi_body,
    grid=(_GRID,),
    in_specs=[
        pl.BlockSpec((_BLK, D), lambda i: (i, 0)),
        pl.BlockSpec((_BLK, D), lambda i: (i, 0)),
        pl.BlockSpec((_BLK, D), lambda i: (i, 0)),
        pl.BlockSpec((_BLK, D), lambda i: (i, 0)),
        pl.BlockSpec((1, D), lambda i: (0, 0)),
    ],
    out_specs=pl.BlockSpec((_BLK, D), lambda i: (i, 0)),
    out_shape=jax.ShapeDtypeStruct((N_PAD, D), jnp.float32),
)


@jax.jit
def kernel(x, edge_index, W1, b1, W2, b2, W3, b3):
    src = edge_index[0]
    dst = edge_index[1]
    pad_e = E_PAD - N_EDGES
    src2 = jnp.concatenate(
        [src, jnp.full((pad_e,), N_NODES, jnp.int32)]).reshape(E_PAD // CHUNK, CHUNK)
    dst2 = jnp.concatenate(
        [dst, jnp.full((pad_e,), N_NODES, jnp.int32)]).reshape(E_PAD // CHUNK, CHUNK)
    x_pad = jnp.concatenate(
        [x, jnp.zeros((N_PAD - N_NODES, D), jnp.float32)], axis=0)

    zeros128 = jnp.zeros((ROWS_PER_TILE, D), jnp.float32)
    ones128 = jnp.ones((CHUNK, D), jnp.float32)
    b1r = b1.reshape(1, D)
    b2r = b2.reshape(1, D)
    b3r = b3.reshape(1, D)

    cnt = _deg_pass(dst2, zeros128, ones128)
    t1 = _mm(x_pad, W1)
    dinv, y1 = _dinv_y(cnt[0], cnt[1], t1)
    p = _msg_pass(y1, src2, dst2, zeros128)
    y2 = _fuse(p[0], p[1], y1, dinv, b1r, W2)
    p = _msg_pass(y2, src2, dst2, zeros128)
    y3 = _fuse(p[0], p[1], y2, dinv, b2r, W3)
    p = _msg_pass(y3, src2, dst2, zeros128)
    out = _epi(p[0], p[1], y3, dinv, b3r)
    return out[:N_NODES]
